# Initial kernel scaffold; baseline (speedup 1.0000x reference)
import jax
import jax.numpy as jnp
from jax.experimental import pallas as pl


def kernel(x, edge_index, edge_attr, W_e1, b_e1, W_n1, b_n1, W_e2, b_e2, W_n2, b_n2):
    raise NotImplementedError("write your pallas kernel here")


# speedup vs baseline: 3.7712x; 3.7712x over previous
import functools

import jax
import jax.numpy as jnp
from jax import lax
from jax.experimental import pallas as pl
from jax.experimental.pallas import tpu as pltpu
from jax.experimental.pallas import tpu_sc as plsc

_N = 10000
_E = 320000
_D = 128
_DE = 16
_ALPHA = 0.5

_NC = 2
_NS = 16
_NW = _NC * _NS
_EPW = _E // _NW
_CH = 80
_NCHUNK = _EPW // _CH
_NP = 10240
_RPS = _NP // _NS

_F32 = jnp.float32
_HIGH = lax.Precision.HIGHEST


def _nodeproj_body(x_ref, w_ref, ps_ref, pd_ref):
    p = jnp.dot(x_ref[...], w_ref[...], precision=_HIGH,
                preferred_element_type=_F32)
    ps_ref[...] = p[:, :_DE]
    pd_ref[...] = p[:, _DE:]


def _nodeproj(x, w_sd, bn_blocks=10):
    bn = _N // bn_blocks
    return pl.pallas_call(
        _nodeproj_body,
        grid=(bn_blocks,),
        in_specs=[
            pl.BlockSpec((bn, _D), lambda i: (i, 0)),
            pl.BlockSpec((_D, 2 * _DE), lambda i: (0, 0)),
        ],
        out_specs=[
            pl.BlockSpec((bn, _DE), lambda i: (i, 0)),
            pl.BlockSpec((bn, _DE), lambda i: (i, 0)),
        ],
        out_shape=[
            jax.ShapeDtypeStruct((_N, _DE), _F32),
            jax.ShapeDtypeStruct((_N, _DE), _F32),
        ],
    )(x, w_sd)


def _edgeproj_body(e_ref, w_ref, b_ref, t_ref):
    t_ref[...] = jnp.dot(e_ref[...], w_ref[...], precision=_HIGH,
                         preferred_element_type=_F32) + b_ref[...]


def _edgeproj(e, w_e, b_e, be_blocks=40):
    bsz = _E // be_blocks
    return pl.pallas_call(
        _edgeproj_body,
        grid=(be_blocks,),
        in_specs=[
            pl.BlockSpec((bsz, _DE), lambda i: (i, 0)),
            pl.BlockSpec((_DE, _DE), lambda i: (0, 0)),
            pl.BlockSpec((1, _DE), lambda i: (0, 0)),
        ],
        out_specs=pl.BlockSpec((bsz, _DE), lambda i: (i, 0)),
        out_shape=jax.ShapeDtypeStruct((_E, _DE), _F32),
    )(e, w_e, b_e.reshape(1, _DE))


def _nodeupd_body(x_ref, agg_ref, wnx_ref, wna_ref, bn_ref, xo_ref):
    agg = agg_ref[0] + agg_ref[1]
    d = (jnp.dot(x_ref[...], wnx_ref[...], precision=_HIGH,
                 preferred_element_type=_F32)
         + jnp.dot(agg, wna_ref[...], precision=_HIGH,
                   preferred_element_type=_F32)
         + bn_ref[...])
    xo_ref[...] = _ALPHA * x_ref[...] + (1.0 - _ALPHA) * jnp.maximum(d, 0.0)


def _nodeupd_proj_body(x_ref, agg_ref, wnx_ref, wna_ref, bn_ref, wsd_ref,
                       xo_ref, ps_ref, pd_ref):
    agg = agg_ref[0] + agg_ref[1]
    d = (jnp.dot(x_ref[...], wnx_ref[...], precision=_HIGH,
                 preferred_element_type=_F32)
         + jnp.dot(agg, wna_ref[...], precision=_HIGH,
                   preferred_element_type=_F32)
         + bn_ref[...])
    xn = _ALPHA * x_ref[...] + (1.0 - _ALPHA) * jnp.maximum(d, 0.0)
    xo_ref[...] = xn
    p = jnp.dot(xn, wsd_ref[...], precision=_HIGH,
                preferred_element_type=_F32)
    ps_ref[...] = p[:, :_DE]
    pd_ref[...] = p[:, _DE:]


def _nodeupd(x, aggp, wn_x, wn_a, b_n, w_sd_next=None, bn_blocks=10):
    bn = _N // bn_blocks
    base_in = [
        pl.BlockSpec((bn, _D), lambda i: (i, 0)),
        pl.BlockSpec((_NC, bn, _DE), lambda i: (0, i, 0)),
        pl.BlockSpec((_D, _D), lambda i: (0, 0)),
        pl.BlockSpec((_DE, _D), lambda i: (0, 0)),
        pl.BlockSpec((1, _D), lambda i: (0, 0)),
    ]
    if w_sd_next is None:
        return pl.pallas_call(
            _nodeupd_body,
            grid=(bn_blocks,),
            in_specs=base_in,
            out_specs=pl.BlockSpec((bn, _D), lambda i: (i, 0)),
            out_shape=jax.ShapeDtypeStruct((_N, _D), _F32),
        )(x, aggp, wn_x, wn_a, b_n.reshape(1, _D))
    return pl.pallas_call(
        _nodeupd_proj_body,
        grid=(bn_blocks,),
        in_specs=base_in + [pl.BlockSpec((_D, 2 * _DE), lambda i: (0, 0))],
        out_specs=[
            pl.BlockSpec((bn, _D), lambda i: (i, 0)),
            pl.BlockSpec((bn, _DE), lambda i: (i, 0)),
            pl.BlockSpec((bn, _DE), lambda i: (i, 0)),
        ],
        out_shape=[
            jax.ShapeDtypeStruct((_N, _D), _F32),
            jax.ShapeDtypeStruct((_N, _DE), _F32),
            jax.ShapeDtypeStruct((_N, _DE), _F32),
        ],
    )(x, aggp, wn_x, wn_a, b_n.reshape(1, _D), w_sd_next)


def _sc_edge_body(ps_hbm, pd_hbm, t_hbm, srcr_hbm, dstr_hbm,
                  enew_hbm, aggp_hbm,
                  src_v, dst_v, ps_b, pd_b, t_b, eo_b, zb, agg_sh,
                  sem_s, sem_d):
    cid = lax.axis_index("c")
    sid = lax.axis_index("s")
    wid = sid * _NC + cid

    def _zrow(i, _):
        zb[i] = jnp.zeros((16,), _F32)
        return 0
    lax.fori_loop(0, _RPS, _zrow, 0)
    pltpu.sync_copy(zb, agg_sh.at[pl.ds(sid * _RPS, _RPS)])
    plsc.subcore_barrier()

    pltpu.sync_copy(srcr_hbm.at[wid], src_v)
    pltpu.sync_copy(dstr_hbm.at[wid], dst_v)

    ebase = wid * _EPW

    def _chunk(j, _):
        cp_s = pltpu.async_copy(ps_hbm.at[src_v.at[j]], ps_b, sem_s)
        cp_d = pltpu.async_copy(pd_hbm.at[dst_v.at[j]], pd_b, sem_d)
        pltpu.sync_copy(t_hbm.at[pl.ds(ebase + j * _CH, _CH)], t_b)
        cp_s.wait()
        cp_d.wait()

        def _row(i, _):
            eo_b[i] = jnp.maximum(ps_b[i] + pd_b[i] + t_b[i], 0.0)
            return 0
        lax.fori_loop(0, _CH, _row, 0)

        pltpu.sync_copy(eo_b, enew_hbm.at[pl.ds(ebase + j * _CH, _CH)])
        pltpu.sync_copy(eo_b, agg_sh.at[dst_v.at[j]], add=True)
        return 0
    lax.fori_loop(0, _NCHUNK, _chunk, 0)

    plsc.subcore_barrier()
    pltpu.sync_copy(agg_sh.at[pl.ds(sid * _RPS, _RPS)],
                    aggp_hbm.at[cid, pl.ds(sid * _RPS, _RPS)])


@functools.cache
def _sc_edge():
  return pl.kernel(
    _sc_edge_body,
    out_type=[
        jax.ShapeDtypeStruct((_E, _DE), _F32),
        jax.ShapeDtypeStruct((_NC, _NP, _DE), _F32),
    ],
    mesh=plsc.VectorSubcoreMesh(core_axis_name="c", subcore_axis_name="s",
                                num_cores=_NC, num_subcores=_NS),
    compiler_params=pltpu.CompilerParams(use_tc_tiling_on_sc=False),
    scratch_types=[
        pltpu.VMEM((_NCHUNK, _CH), jnp.int32),
        pltpu.VMEM((_NCHUNK, _CH), jnp.int32),
        pltpu.VMEM((_CH, _DE), _F32),
        pltpu.VMEM((_CH, _DE), _F32),
        pltpu.VMEM((_CH, _DE), _F32),
        pltpu.VMEM((_CH, _DE), _F32),
        pltpu.VMEM((_RPS, _DE), _F32),
        pltpu.VMEM_SHARED((_NP, _DE), _F32),
        pltpu.SemaphoreType.DMA,
        pltpu.SemaphoreType.DMA,
    ],
  )


def kernel(x, edge_index, edge_attr, W_e1, b_e1, W_n1, b_n1,
           W_e2, b_e2, W_n2, b_n2):
    src = edge_index[0].reshape(_NW, _NCHUNK, _CH)
    dst = edge_index[1].reshape(_NW, _NCHUNK, _CH)

    wsd1 = jnp.concatenate([W_e1[:_D], W_e1[_D:2 * _D]], axis=1)
    we1 = W_e1[2 * _D:]
    wsd2 = jnp.concatenate([W_e2[:_D], W_e2[_D:2 * _D]], axis=1)
    we2 = W_e2[2 * _D:]
    wn1_x, wn1_a = W_n1[:_D], W_n1[_D:]
    wn2_x, wn2_a = W_n2[:_D], W_n2[_D:]

    ps1, pd1 = _nodeproj(x, wsd1)
    t1 = _edgeproj(edge_attr, we1, b_e1)
    e1, aggp1 = _sc_edge()(ps1, pd1, t1, src, dst)
    x1, ps2, pd2 = _nodeupd(x, aggp1[:, :_N], wn1_x, wn1_a, b_n1,
                            w_sd_next=wsd2)

    t2 = _edgeproj(e1, we2, b_e2)
    e2, aggp2 = _sc_edge()(ps2, pd2, t2, src, dst)
    x2 = _nodeupd(x1, aggp2[:, :_N], wn2_x, wn2_a, b_n2)

    return (x2, e2, (edge_attr, e1, e2))


# speedup vs baseline: 4.2327x; 1.1224x over previous
import functools

import jax
import jax.numpy as jnp
from jax import lax
from jax.experimental import pallas as pl
from jax.experimental.pallas import tpu as pltpu
from jax.experimental.pallas import tpu_sc as plsc

_N = 10000
_E = 320000
_D = 128
_DE = 16
_ALPHA = 0.5

_NC = 2
_NS = 16
_NW = _NC * _NS
_EPW = _E // _NW
_CH = 80
_NCHUNK = _EPW // _CH
_NP = 10240
_RPS = _NP // _NS

_F32 = jnp.float32
_HIGH = lax.Precision.HIGHEST


def _nodeproj_body(x_ref, w_ref, ps_ref, pd_ref):
    p = jnp.dot(x_ref[...], w_ref[...], precision=_HIGH,
                preferred_element_type=_F32)
    ps_ref[...] = p[:, :_DE]
    pd_ref[...] = p[:, _DE:]


def _nodeproj(x, w_sd, bn_blocks=10):
    bn = _N // bn_blocks
    return pl.pallas_call(
        _nodeproj_body,
        grid=(bn_blocks,),
        in_specs=[
            pl.BlockSpec((bn, _D), lambda i: (i, 0)),
            pl.BlockSpec((_D, 2 * _DE), lambda i: (0, 0)),
        ],
        out_specs=[
            pl.BlockSpec((bn, _DE), lambda i: (i, 0)),
            pl.BlockSpec((bn, _DE), lambda i: (i, 0)),
        ],
        out_shape=[
            jax.ShapeDtypeStruct((_N, _DE), _F32),
            jax.ShapeDtypeStruct((_N, _DE), _F32),
        ],
    )(x, w_sd)


def _edgeproj_body(e_ref, w_ref, b_ref, t_ref):
    t_ref[...] = jnp.dot(e_ref[...], w_ref[...], precision=_HIGH,
                         preferred_element_type=_F32) + b_ref[...]


def _edgeproj(e, w_e, b_e, be_blocks=40):
    bsz = _E // be_blocks
    return pl.pallas_call(
        _edgeproj_body,
        grid=(be_blocks,),
        in_specs=[
            pl.BlockSpec((bsz, _DE), lambda i: (i, 0)),
            pl.BlockSpec((_DE, _DE), lambda i: (0, 0)),
            pl.BlockSpec((1, _DE), lambda i: (0, 0)),
        ],
        out_specs=pl.BlockSpec((bsz, _DE), lambda i: (i, 0)),
        out_shape=jax.ShapeDtypeStruct((_E, _DE), _F32),
    )(e, w_e, b_e.reshape(1, _DE))


def _nodeupd_body(x_ref, agg_ref, wnx_ref, wna_ref, bn_ref, xo_ref):
    agg = agg_ref[0] + agg_ref[1]
    d = (jnp.dot(x_ref[...], wnx_ref[...], precision=_HIGH,
                 preferred_element_type=_F32)
         + jnp.dot(agg, wna_ref[...], precision=_HIGH,
                   preferred_element_type=_F32)
         + bn_ref[...])
    xo_ref[...] = _ALPHA * x_ref[...] + (1.0 - _ALPHA) * jnp.maximum(d, 0.0)


def _nodeupd_proj_body(x_ref, agg_ref, wnx_ref, wna_ref, bn_ref, wsd_ref,
                       xo_ref, ps_ref, pd_ref):
    agg = agg_ref[0] + agg_ref[1]
    d = (jnp.dot(x_ref[...], wnx_ref[...], precision=_HIGH,
                 preferred_element_type=_F32)
         + jnp.dot(agg, wna_ref[...], precision=_HIGH,
                   preferred_element_type=_F32)
         + bn_ref[...])
    xn = _ALPHA * x_ref[...] + (1.0 - _ALPHA) * jnp.maximum(d, 0.0)
    xo_ref[...] = xn
    p = jnp.dot(xn, wsd_ref[...], precision=_HIGH,
                preferred_element_type=_F32)
    ps_ref[...] = p[:, :_DE]
    pd_ref[...] = p[:, _DE:]


def _nodeupd(x, aggp, wn_x, wn_a, b_n, w_sd_next=None, bn_blocks=10):
    bn = _N // bn_blocks
    base_in = [
        pl.BlockSpec((bn, _D), lambda i: (i, 0)),
        pl.BlockSpec((_NC, bn, _DE), lambda i: (0, i, 0)),
        pl.BlockSpec((_D, _D), lambda i: (0, 0)),
        pl.BlockSpec((_DE, _D), lambda i: (0, 0)),
        pl.BlockSpec((1, _D), lambda i: (0, 0)),
    ]
    if w_sd_next is None:
        return pl.pallas_call(
            _nodeupd_body,
            grid=(bn_blocks,),
            in_specs=base_in,
            out_specs=pl.BlockSpec((bn, _D), lambda i: (i, 0)),
            out_shape=jax.ShapeDtypeStruct((_N, _D), _F32),
        )(x, aggp, wn_x, wn_a, b_n.reshape(1, _D))
    return pl.pallas_call(
        _nodeupd_proj_body,
        grid=(bn_blocks,),
        in_specs=base_in + [pl.BlockSpec((_D, 2 * _DE), lambda i: (0, 0))],
        out_specs=[
            pl.BlockSpec((bn, _D), lambda i: (i, 0)),
            pl.BlockSpec((bn, _DE), lambda i: (i, 0)),
            pl.BlockSpec((bn, _DE), lambda i: (i, 0)),
        ],
        out_shape=[
            jax.ShapeDtypeStruct((_N, _D), _F32),
            jax.ShapeDtypeStruct((_N, _DE), _F32),
            jax.ShapeDtypeStruct((_N, _DE), _F32),
        ],
    )(x, aggp, wn_x, wn_a, b_n.reshape(1, _D), w_sd_next)


_NBUF = 2


def _sc_edge_body(ps_hbm, pd_hbm, t_hbm, srcr_hbm, dstr_hbm,
                  enew_hbm, aggp_hbm,
                  src_v, dst_v, ps_b, pd_b, t_b, eo_b, zb, agg_sh,
                  sem_g, sem_w):
    cid = lax.axis_index("c")
    sid = lax.axis_index("s")
    wid = sid * _NC + cid

    def _zrow(i, _):
        zb[i] = jnp.zeros((16,), _F32)
        return 0
    lax.fori_loop(0, _RPS, _zrow, 0)
    pltpu.sync_copy(zb, agg_sh.at[pl.ds(sid * _RPS, _RPS)])
    plsc.subcore_barrier()

    pltpu.sync_copy(srcr_hbm.at[wid], src_v)
    pltpu.sync_copy(dstr_hbm.at[wid], dst_v)

    ebase = wid * _EPW

    def _fire(j, p, sem):
        return (
            pltpu.async_copy(ps_hbm.at[src_v.at[j]], ps_b.at[p], sem),
            pltpu.async_copy(pd_hbm.at[dst_v.at[j]], pd_b.at[p], sem),
            pltpu.async_copy(t_hbm.at[pl.ds(ebase + j * _CH, _CH)],
                             t_b.at[p], sem),
        )

    def _consume(j, p, cps):
        for cp in cps:
            cp.wait()

        def _row(r, _):
            eo_b[p, r] = jnp.maximum(
                ps_b[p, r] + pd_b[p, r] + t_b[p, r], 0.0)
            return 0
        lax.fori_loop(0, _CH, _row, 0)
        pltpu.sync_copy(eo_b.at[p], enew_hbm.at[pl.ds(ebase + j * _CH, _CH)])
        pltpu.sync_copy(eo_b.at[p], agg_sh.at[dst_v.at[j]], add=True)

    cps0 = _fire(0, 0, sem_g)

    def _round(i, _):
        j0 = 2 * i
        cps1 = _fire(j0 + 1, 1, sem_w)
        _consume(j0, 0, cps0)
        _fire(j0 + 2, 0, sem_g)
        _consume(j0 + 1, 1, cps1)
        return 0
    lax.fori_loop(0, (_NCHUNK - 1) // 2, _round, 0)
    _consume(_NCHUNK - 1, 0, cps0)

    plsc.subcore_barrier()
    pltpu.sync_copy(agg_sh.at[pl.ds(sid * _RPS, _RPS)],
                    aggp_hbm.at[cid, pl.ds(sid * _RPS, _RPS)])


@functools.cache
def _sc_edge():
  return pl.kernel(
    _sc_edge_body,
    out_type=[
        jax.ShapeDtypeStruct((_E, _DE), _F32),
        jax.ShapeDtypeStruct((_NC, _NP, _DE), _F32),
    ],
    mesh=plsc.VectorSubcoreMesh(core_axis_name="c", subcore_axis_name="s",
                                num_cores=_NC, num_subcores=_NS),
    compiler_params=pltpu.CompilerParams(use_tc_tiling_on_sc=False),
    scratch_types=[
        pltpu.VMEM((_NCHUNK, _CH), jnp.int32),
        pltpu.VMEM((_NCHUNK, _CH), jnp.int32),
        pltpu.VMEM((_NBUF, _CH, _DE), _F32),
        pltpu.VMEM((_NBUF, _CH, _DE), _F32),
        pltpu.VMEM((_NBUF, _CH, _DE), _F32),
        pltpu.VMEM((_NBUF, _CH, _DE), _F32),
        pltpu.VMEM((_RPS, _DE), _F32),
        pltpu.VMEM_SHARED((_NP, _DE), _F32),
        pltpu.SemaphoreType.DMA,
        pltpu.SemaphoreType.DMA,
    ],
  )


def kernel(x, edge_index, edge_attr, W_e1, b_e1, W_n1, b_n1,
           W_e2, b_e2, W_n2, b_n2):
    src = edge_index[0].reshape(_NW, _NCHUNK, _CH)
    dst = edge_index[1].reshape(_NW, _NCHUNK, _CH)

    wsd1 = jnp.concatenate([W_e1[:_D], W_e1[_D:2 * _D]], axis=1)
    we1 = W_e1[2 * _D:]
    wsd2 = jnp.concatenate([W_e2[:_D], W_e2[_D:2 * _D]], axis=1)
    we2 = W_e2[2 * _D:]
    wn1_x, wn1_a = W_n1[:_D], W_n1[_D:]
    wn2_x, wn2_a = W_n2[:_D], W_n2[_D:]

    ps1, pd1 = _nodeproj(x, wsd1)
    t1 = _edgeproj(edge_attr, we1, b_e1)
    e1, aggp1 = _sc_edge()(ps1, pd1, t1, src, dst)
    x1, ps2, pd2 = _nodeupd(x, aggp1, wn1_x, wn1_a, b_n1, w_sd_next=wsd2)

    t2 = _edgeproj(e1, we2, b_e2)
    e2, aggp2 = _sc_edge()(ps2, pd2, t2, src, dst)
    x2 = _nodeupd(x1, aggp2, wn2_x, wn2_a, b_n2)

    return (x2, e2, (edge_attr, e1, e2))


# speedup vs baseline: 6.4975x; 1.5351x over previous
import functools

import jax
import jax.numpy as jnp
from jax import lax
from jax.experimental import pallas as pl
from jax.experimental.pallas import tpu as pltpu
from jax.experimental.pallas import tpu_sc as plsc

_N = 10000
_E = 320000
_D = 128
_DE = 16
_ALPHA = 0.5

_NC = 2
_NS = 16
_NW = _NC * _NS
_EPW = _E // _NW
_CH = 80
_NCHUNK = _EPW // _CH
_NP = 10240
_RPS = _NP // _NS
_E8 = _E // 8
_CH8 = _CH // 8

_F32 = jnp.float32
_HIGH = lax.Precision.HIGHEST


def _nodeproj_body(x_ref, w_ref, ps_ref, pd_ref):
    p = jnp.dot(x_ref[...], w_ref[...], precision=_HIGH,
                preferred_element_type=_F32)
    ps_ref[...] = p[:, :_DE]
    pd_ref[...] = p[:, _DE:]


def _nodeproj(x, w_sd, bn_blocks=10):
    bn = _N // bn_blocks
    return pl.pallas_call(
        _nodeproj_body,
        grid=(bn_blocks,),
        in_specs=[
            pl.BlockSpec((bn, _D), lambda i: (i, 0)),
            pl.BlockSpec((_D, 2 * _DE), lambda i: (0, 0)),
        ],
        out_specs=[
            pl.BlockSpec((bn, _DE), lambda i: (i, 0)),
            pl.BlockSpec((bn, _DE), lambda i: (i, 0)),
        ],
        out_shape=[
            jax.ShapeDtypeStruct((_N, _DE), _F32),
            jax.ShapeDtypeStruct((_N, _DE), _F32),
        ],
    )(x, w_sd)


def _edgeproj_body(e_ref, w_ref, b_ref, t_ref):
    t_ref[...] = jnp.dot(e_ref[...], w_ref[...], precision=_HIGH,
                         preferred_element_type=_F32) + b_ref[...]


def _edgeproj(e128, w_bd, b128, be_blocks=40):
    bsz = _E8 // be_blocks
    return pl.pallas_call(
        _edgeproj_body,
        grid=(be_blocks,),
        in_specs=[
            pl.BlockSpec((bsz, 128), lambda i: (i, 0)),
            pl.BlockSpec((128, 128), lambda i: (0, 0)),
            pl.BlockSpec((1, 128), lambda i: (0, 0)),
        ],
        out_specs=pl.BlockSpec((bsz, 128), lambda i: (i, 0)),
        out_shape=jax.ShapeDtypeStruct((_E8, 128), _F32),
    )(e128, w_bd, b128)


def _nodeupd_body(x_ref, agg_ref, wnx_ref, wna_ref, bn_ref, xo_ref):
    agg = agg_ref[0] + agg_ref[1]
    d = (jnp.dot(x_ref[...], wnx_ref[...], precision=_HIGH,
                 preferred_element_type=_F32)
         + jnp.dot(agg, wna_ref[...], precision=_HIGH,
                   preferred_element_type=_F32)
         + bn_ref[...])
    xo_ref[...] = _ALPHA * x_ref[...] + (1.0 - _ALPHA) * jnp.maximum(d, 0.0)


def _nodeupd_proj_body(x_ref, agg_ref, wnx_ref, wna_ref, bn_ref, wsd_ref,
                       xo_ref, ps_ref, pd_ref):
    agg = agg_ref[0] + agg_ref[1]
    d = (jnp.dot(x_ref[...], wnx_ref[...], precision=_HIGH,
                 preferred_element_type=_F32)
         + jnp.dot(agg, wna_ref[...], precision=_HIGH,
                   preferred_element_type=_F32)
         + bn_ref[...])
    xn = _ALPHA * x_ref[...] + (1.0 - _ALPHA) * jnp.maximum(d, 0.0)
    xo_ref[...] = xn
    p = jnp.dot(xn, wsd_ref[...], precision=_HIGH,
                preferred_element_type=_F32)
    ps_ref[...] = p[:, :_DE]
    pd_ref[...] = p[:, _DE:]


def _nodeupd(x, aggp, wn_x, wn_a, b_n, w_sd_next=None, bn_blocks=10):
    bn = _N // bn_blocks
    base_in = [
        pl.BlockSpec((bn, _D), lambda i: (i, 0)),
        pl.BlockSpec((_NC, bn, _DE), lambda i: (0, i, 0)),
        pl.BlockSpec((_D, _D), lambda i: (0, 0)),
        pl.BlockSpec((_DE, _D), lambda i: (0, 0)),
        pl.BlockSpec((1, _D), lambda i: (0, 0)),
    ]
    if w_sd_next is None:
        return pl.pallas_call(
            _nodeupd_body,
            grid=(bn_blocks,),
            in_specs=base_in,
            out_specs=pl.BlockSpec((bn, _D), lambda i: (i, 0)),
            out_shape=jax.ShapeDtypeStruct((_N, _D), _F32),
        )(x, aggp, wn_x, wn_a, b_n.reshape(1, _D))
    return pl.pallas_call(
        _nodeupd_proj_body,
        grid=(bn_blocks,),
        in_specs=base_in + [pl.BlockSpec((_D, 2 * _DE), lambda i: (0, 0))],
        out_specs=[
            pl.BlockSpec((bn, _D), lambda i: (i, 0)),
            pl.BlockSpec((bn, _DE), lambda i: (i, 0)),
            pl.BlockSpec((bn, _DE), lambda i: (i, 0)),
        ],
        out_shape=[
            jax.ShapeDtypeStruct((_N, _D), _F32),
            jax.ShapeDtypeStruct((_N, _DE), _F32),
            jax.ShapeDtypeStruct((_N, _DE), _F32),
        ],
    )(x, aggp, wn_x, wn_a, b_n.reshape(1, _D), w_sd_next)


_NBUF = 2


def _sc_edge_body(ps_hbm, pd_hbm, t_hbm, srcr_hbm, dstr_hbm,
                  enew_hbm, aggp_hbm,
                  src_v, dst_v, ps_b, pd_b, t_b, eo_b, zb, agg_sh,
                  sem_g, sem_w):
    cid = lax.axis_index("c")
    sid = lax.axis_index("s")
    wid = sid * _NC + cid

    def _zrow(i, _):
        zb[i] = jnp.zeros((16,), _F32)
        return 0
    lax.fori_loop(0, _RPS, _zrow, 0)
    pltpu.sync_copy(zb, agg_sh.at[pl.ds(sid * _RPS, _RPS)])
    plsc.subcore_barrier()

    pltpu.sync_copy(srcr_hbm.at[wid], src_v)
    pltpu.sync_copy(dstr_hbm.at[wid], dst_v)

    ebase = wid * _EPW

    def _fire(j, p, sem):
        return (
            pltpu.async_copy(ps_hbm.at[src_v.at[j]], ps_b.at[p], sem),
            pltpu.async_copy(pd_hbm.at[dst_v.at[j]], pd_b.at[p], sem),
            pltpu.async_copy(t_hbm.at[pl.ds(ebase // 8 + j * _CH8, _CH8)],
                             t_b.at[p], sem),
        )

    def _consume(j, p, cps):
        for cp in cps:
            cp.wait()

        def _row(rr, _):
            for a in range(8):
                r = rr * 8 + a
                eo_b[p, r] = jnp.maximum(
                    ps_b[p, r] + pd_b[p, r]
                    + t_b[p, rr, pl.ds(a * _DE, _DE)], 0.0)
            return 0
        lax.fori_loop(0, _CH8, _row, 0)
        pltpu.sync_copy(eo_b.at[p], enew_hbm.at[pl.ds(ebase + j * _CH, _CH)])
        pltpu.sync_copy(eo_b.at[p], agg_sh.at[dst_v.at[j]], add=True)

    cps0 = _fire(0, 0, sem_g)

    def _round(i, _):
        j0 = 2 * i
        cps1 = _fire(j0 + 1, 1, sem_w)
        _consume(j0, 0, cps0)
        _fire(j0 + 2, 0, sem_g)
        _consume(j0 + 1, 1, cps1)
        return 0
    lax.fori_loop(0, (_NCHUNK - 1) // 2, _round, 0)
    _consume(_NCHUNK - 1, 0, cps0)

    plsc.subcore_barrier()
    pltpu.sync_copy(agg_sh.at[pl.ds(sid * _RPS, _RPS)],
                    aggp_hbm.at[cid, pl.ds(sid * _RPS, _RPS)])


@functools.cache
def _sc_edge():
  return pl.kernel(
    _sc_edge_body,
    out_type=[
        jax.ShapeDtypeStruct((_E, _DE), _F32),
        jax.ShapeDtypeStruct((_NC, _NP, _DE), _F32),
    ],
    mesh=plsc.VectorSubcoreMesh(core_axis_name="c", subcore_axis_name="s",
                                num_cores=_NC, num_subcores=_NS),
    compiler_params=pltpu.CompilerParams(use_tc_tiling_on_sc=False),
    scratch_types=[
        pltpu.VMEM((_NCHUNK, _CH), jnp.int32),
        pltpu.VMEM((_NCHUNK, _CH), jnp.int32),
        pltpu.VMEM((_NBUF, _CH, _DE), _F32),
        pltpu.VMEM((_NBUF, _CH, _DE), _F32),
        pltpu.VMEM((_NBUF, _CH8, 128), _F32),
        pltpu.VMEM((_NBUF, _CH, _DE), _F32),
        pltpu.VMEM((_RPS, _DE), _F32),
        pltpu.VMEM_SHARED((_NP, _DE), _F32),
        pltpu.SemaphoreType.DMA,
        pltpu.SemaphoreType.DMA,
    ],
  )


def kernel(x, edge_index, edge_attr, W_e1, b_e1, W_n1, b_n1,
           W_e2, b_e2, W_n2, b_n2):
    src = edge_index[0].reshape(_NW, _NCHUNK, _CH)
    dst = edge_index[1].reshape(_NW, _NCHUNK, _CH)

    wsd1 = jnp.concatenate([W_e1[:_D], W_e1[_D:2 * _D]], axis=1)
    we1 = W_e1[2 * _D:]
    wsd2 = jnp.concatenate([W_e2[:_D], W_e2[_D:2 * _D]], axis=1)
    we2 = W_e2[2 * _D:]
    wn1_x, wn1_a = W_n1[:_D], W_n1[_D:]
    wn2_x, wn2_a = W_n2[:_D], W_n2[_D:]

    eye8 = jnp.eye(8, dtype=_F32)
    bd1 = jnp.kron(eye8, we1)
    bd2 = jnp.kron(eye8, we2)
    b1_128 = jnp.tile(b_e1.reshape(1, _DE), (1, 8))
    b2_128 = jnp.tile(b_e2.reshape(1, _DE), (1, 8))

    ps1, pd1 = _nodeproj(x, wsd1)
    t1 = _edgeproj(edge_attr.reshape(_E8, 128), bd1, b1_128)
    e1, aggp1 = _sc_edge()(ps1, pd1, t1, src, dst)
    x1, ps2, pd2 = _nodeupd(x, aggp1, wn1_x, wn1_a, b_n1, w_sd_next=wsd2)

    t2 = _edgeproj(e1.reshape(_E8, 128), bd2, b2_128)
    e2, aggp2 = _sc_edge()(ps2, pd2, t2, src, dst)
    x2 = _nodeupd(x1, aggp2, wn2_x, wn2_a, b_n2)

    return (x2, e2, (edge_attr, e1, e2))


# speedup vs baseline: 7.2626x; 1.1177x over previous
import functools

import jax
import jax.numpy as jnp
from jax import lax
from jax.experimental import pallas as pl
from jax.experimental.pallas import tpu as pltpu
from jax.experimental.pallas import tpu_sc as plsc

_N = 10000
_E = 320000
_D = 128
_DE = 16
_ALPHA = 0.5

_NC = 2
_NS = 16
_NW = _NC * _NS
_EPW = _E // _NW
_CH = 80
_NCHUNK = _EPW // _CH
_NP = 10240
_RPS = _NP // _NS
_E8 = _E // 8
_CH8 = _CH // 8

_F32 = jnp.float32
_HIGH = lax.Precision.HIGHEST

_EB = 25
_EBS = _E8 // _EB
_NBS = _N // _EB


def _proj1_body(e_ref, wbd_ref, b_ref, x_ref, wsd_ref, t_ref, ps_ref, pd_ref):
    t_ref[...] = jnp.dot(e_ref[...], wbd_ref[...], precision=_HIGH,
                         preferred_element_type=_F32) + b_ref[...]
    p = jnp.dot(x_ref[...], wsd_ref[...], precision=_HIGH,
                preferred_element_type=_F32)
    ps_ref[...] = p[:, :_DE]
    pd_ref[...] = p[:, _DE:]


def _proj1(e128, w_bd, b128, x, w_sd):
    return pl.pallas_call(
        _proj1_body,
        grid=(_EB,),
        in_specs=[
            pl.BlockSpec((_EBS, 128), lambda i: (i, 0)),
            pl.BlockSpec((128, 128), lambda i: (0, 0)),
            pl.BlockSpec((1, 128), lambda i: (0, 0)),
            pl.BlockSpec((_NBS, _D), lambda i: (i, 0)),
            pl.BlockSpec((_D, 2 * _DE), lambda i: (0, 0)),
        ],
        out_specs=[
            pl.BlockSpec((_EBS, 128), lambda i: (i, 0)),
            pl.BlockSpec((_NBS, _DE), lambda i: (i, 0)),
            pl.BlockSpec((_NBS, _DE), lambda i: (i, 0)),
        ],
        out_shape=[
            jax.ShapeDtypeStruct((_E8, 128), _F32),
            jax.ShapeDtypeStruct((_N, _DE), _F32),
            jax.ShapeDtypeStruct((_N, _DE), _F32),
        ],
    )(e128, w_bd, b128, x, w_sd)


def _upd2_body(e_ref, wbd_ref, b_ref, x_ref, agg_ref, wnx_ref, wna_ref,
               bn_ref, wsd_ref, t_ref, xo_ref, ps_ref, pd_ref):
    t_ref[...] = jnp.dot(e_ref[...], wbd_ref[...], precision=_HIGH,
                         preferred_element_type=_F32) + b_ref[...]
    agg = agg_ref[0] + agg_ref[1]
    d = (jnp.dot(x_ref[...], wnx_ref[...], precision=_HIGH,
                 preferred_element_type=_F32)
         + jnp.dot(agg, wna_ref[...], precision=_HIGH,
                   preferred_element_type=_F32)
         + bn_ref[...])
    xn = _ALPHA * x_ref[...] + (1.0 - _ALPHA) * jnp.maximum(d, 0.0)
    xo_ref[...] = xn
    p = jnp.dot(xn, wsd_ref[...], precision=_HIGH,
                preferred_element_type=_F32)
    ps_ref[...] = p[:, :_DE]
    pd_ref[...] = p[:, _DE:]


def _upd2(e128, w_bd, b128, x, aggp, wn_x, wn_a, b_n, w_sd):
    return pl.pallas_call(
        _upd2_body,
        grid=(_EB,),
        in_specs=[
            pl.BlockSpec((_EBS, 128), lambda i: (i, 0)),
            pl.BlockSpec((128, 128), lambda i: (0, 0)),
            pl.BlockSpec((1, 128), lambda i: (0, 0)),
            pl.BlockSpec((_NBS, _D), lambda i: (i, 0)),
            pl.BlockSpec((_NC, _NBS, _DE), lambda i: (0, i, 0)),
            pl.BlockSpec((_D, _D), lambda i: (0, 0)),
            pl.BlockSpec((_DE, _D), lambda i: (0, 0)),
            pl.BlockSpec((1, _D), lambda i: (0, 0)),
            pl.BlockSpec((_D, 2 * _DE), lambda i: (0, 0)),
        ],
        out_specs=[
            pl.BlockSpec((_EBS, 128), lambda i: (i, 0)),
            pl.BlockSpec((_NBS, _D), lambda i: (i, 0)),
            pl.BlockSpec((_NBS, _DE), lambda i: (i, 0)),
            pl.BlockSpec((_NBS, _DE), lambda i: (i, 0)),
        ],
        out_shape=[
            jax.ShapeDtypeStruct((_E8, 128), _F32),
            jax.ShapeDtypeStruct((_N, _D), _F32),
            jax.ShapeDtypeStruct((_N, _DE), _F32),
            jax.ShapeDtypeStruct((_N, _DE), _F32),
        ],
    )(e128, w_bd, b128, x, aggp, wn_x, wn_a, b_n.reshape(1, _D), w_sd)


def _nodeupd_body(x_ref, agg_ref, wnx_ref, wna_ref, bn_ref, xo_ref):
    agg = agg_ref[0] + agg_ref[1]
    d = (jnp.dot(x_ref[...], wnx_ref[...], precision=_HIGH,
                 preferred_element_type=_F32)
         + jnp.dot(agg, wna_ref[...], precision=_HIGH,
                   preferred_element_type=_F32)
         + bn_ref[...])
    xo_ref[...] = _ALPHA * x_ref[...] + (1.0 - _ALPHA) * jnp.maximum(d, 0.0)


def _nodeupd(x, aggp, wn_x, wn_a, b_n, bn_blocks=10):
    bn = _N // bn_blocks
    return pl.pallas_call(
        _nodeupd_body,
        grid=(bn_blocks,),
        in_specs=[
            pl.BlockSpec((bn, _D), lambda i: (i, 0)),
            pl.BlockSpec((_NC, bn, _DE), lambda i: (0, i, 0)),
            pl.BlockSpec((_D, _D), lambda i: (0, 0)),
            pl.BlockSpec((_DE, _D), lambda i: (0, 0)),
            pl.BlockSpec((1, _D), lambda i: (0, 0)),
        ],
        out_specs=pl.BlockSpec((bn, _D), lambda i: (i, 0)),
        out_shape=jax.ShapeDtypeStruct((_N, _D), _F32),
    )(x, aggp, wn_x, wn_a, b_n.reshape(1, _D))


_NBUF = 2


def _sc_edge_body(ps_hbm, pd_hbm, t_hbm, srcr_hbm, dstr_hbm,
                  enew_hbm, aggp_hbm,
                  src_v, dst_v, ps_b, pd_b, t_b, eo_b, zb, agg_sh,
                  sem_ga, sem_gb, sem_wa, sem_wb):
    cid = lax.axis_index("c")
    sid = lax.axis_index("s")
    wid = sid * _NC + cid

    def _zrow(i, _):
        zb[i] = jnp.zeros((16,), _F32)
        return 0
    lax.fori_loop(0, _RPS, _zrow, 0)
    pltpu.sync_copy(zb, agg_sh.at[pl.ds(sid * _RPS, _RPS)])
    plsc.subcore_barrier()

    pltpu.sync_copy(srcr_hbm.at[wid], src_v)
    pltpu.sync_copy(dstr_hbm.at[wid], dst_v)

    ebase = wid * _EPW
    sem_w = (sem_wa, sem_wb)

    def _fire(j, p, sem):
        return (
            pltpu.async_copy(ps_hbm.at[src_v.at[j]], ps_b.at[p], sem),
            pltpu.async_copy(pd_hbm.at[dst_v.at[j]], pd_b.at[p], sem),
            pltpu.async_copy(t_hbm.at[pl.ds(ebase // 8 + j * _CH8, _CH8)],
                             t_b.at[p], sem),
        )

    def _drain_w(p):
        pltpu.make_async_copy(
            enew_hbm.at[pl.ds(0, _CH)], eo_b.at[p], sem_w[p]).wait()

    def _consume(j, p, cps):
        for cp in cps:
            cp.wait()

        @pl.when(j >= _NBUF)
        def _():
            _drain_w(p)

        def _row(rr, _):
            for a in range(8):
                r = rr * 8 + a
                eo_b[p, r] = jnp.maximum(
                    ps_b[p, r] + pd_b[p, r]
                    + t_b[p, rr, pl.ds(a * _DE, _DE)], 0.0)
            return 0
        lax.fori_loop(0, _CH8, _row, 0)
        pltpu.sync_copy(eo_b.at[p], agg_sh.at[dst_v.at[j]], add=True)
        pltpu.async_copy(eo_b.at[p], enew_hbm.at[pl.ds(ebase + j * _CH, _CH)],
                         sem_w[p])

    cps0 = _fire(0, 0, sem_ga)

    def _round(i, _):
        j0 = 2 * i
        cps1 = _fire(j0 + 1, 1, sem_gb)
        _consume(j0, 0, cps0)
        _fire(j0 + 2, 0, sem_ga)
        _consume(j0 + 1, 1, cps1)
        return 0
    lax.fori_loop(0, (_NCHUNK - 1) // 2, _round, 0)
    _consume(_NCHUNK - 1, 0, cps0)
    _drain_w(0)
    _drain_w(1)

    plsc.subcore_barrier()
    pltpu.sync_copy(agg_sh.at[pl.ds(sid * _RPS, _RPS)],
                    aggp_hbm.at[cid, pl.ds(sid * _RPS, _RPS)])


@functools.cache
def _sc_edge():
  return pl.kernel(
    _sc_edge_body,
    out_type=[
        jax.ShapeDtypeStruct((_E, _DE), _F32),
        jax.ShapeDtypeStruct((_NC, _NP, _DE), _F32),
    ],
    mesh=plsc.VectorSubcoreMesh(core_axis_name="c", subcore_axis_name="s",
                                num_cores=_NC, num_subcores=_NS),
    compiler_params=pltpu.CompilerParams(use_tc_tiling_on_sc=False),
    scratch_types=[
        pltpu.VMEM((_NCHUNK, _CH), jnp.int32),
        pltpu.VMEM((_NCHUNK, _CH), jnp.int32),
        pltpu.VMEM((_NBUF, _CH, _DE), _F32),
        pltpu.VMEM((_NBUF, _CH, _DE), _F32),
        pltpu.VMEM((_NBUF, _CH8, 128), _F32),
        pltpu.VMEM((_NBUF, _CH, _DE), _F32),
        pltpu.VMEM((_RPS, _DE), _F32),
        pltpu.VMEM_SHARED((_NP, _DE), _F32),
        pltpu.SemaphoreType.DMA,
        pltpu.SemaphoreType.DMA,
        pltpu.SemaphoreType.DMA,
        pltpu.SemaphoreType.DMA,
    ],
  )


def kernel(x, edge_index, edge_attr, W_e1, b_e1, W_n1, b_n1,
           W_e2, b_e2, W_n2, b_n2):
    src = edge_index[0].reshape(_NW, _NCHUNK, _CH)
    dst = edge_index[1].reshape(_NW, _NCHUNK, _CH)

    wsd1 = jnp.concatenate([W_e1[:_D], W_e1[_D:2 * _D]], axis=1)
    we1 = W_e1[2 * _D:]
    wsd2 = jnp.concatenate([W_e2[:_D], W_e2[_D:2 * _D]], axis=1)
    we2 = W_e2[2 * _D:]
    wn1_x, wn1_a = W_n1[:_D], W_n1[_D:]
    wn2_x, wn2_a = W_n2[:_D], W_n2[_D:]

    eye8 = jnp.eye(8, dtype=_F32)
    bd1 = jnp.kron(eye8, we1)
    bd2 = jnp.kron(eye8, we2)
    b1_128 = jnp.tile(b_e1.reshape(1, _DE), (1, 8))
    b2_128 = jnp.tile(b_e2.reshape(1, _DE), (1, 8))

    t1, ps1, pd1 = _proj1(edge_attr.reshape(_E8, 128), bd1, b1_128, x, wsd1)
    e1, aggp1 = _sc_edge()(ps1, pd1, t1, src, dst)

    t2, x1, ps2, pd2 = _upd2(e1.reshape(_E8, 128), bd2, b2_128, x, aggp1,
                             wn1_x, wn1_a, b_n1, wsd2)
    e2, aggp2 = _sc_edge()(ps2, pd2, t2, src, dst)
    x2 = _nodeupd(x1, aggp2, wn2_x, wn2_a, b_n2)

    return (x2, e2, (edge_attr, e1, e2))


# speedup vs baseline: 7.5952x; 1.0458x over previous
import functools

import jax
import jax.numpy as jnp
from jax import lax
from jax.experimental import pallas as pl
from jax.experimental.pallas import tpu as pltpu
from jax.experimental.pallas import tpu_sc as plsc

_N = 10000
_E = 320000
_D = 128
_DE = 16
_ALPHA = 0.5

_NC = 2
_NS = 16
_NW = _NC * _NS
_CH = 128
_NCHUNK = _E // _CH
_MSTEADY = _NCHUNK // _NW
_NWEXTRA = _NCHUNK - _MSTEADY * _NW
_NP = 10240
_RPS = _NP // _NS
_E8 = _E // 8
_CH8 = _CH // 8

_F32 = jnp.float32
_HIGH = lax.Precision.HIGHEST

_EB = 25
_EBS = _E8 // _EB
_NBS = _N // _EB


def _proj1_body(e_ref, wbd_ref, b_ref, x_ref, wsd_ref, t_ref, ps_ref, pd_ref):
    t_ref[...] = jnp.dot(e_ref[...], wbd_ref[...], precision=_HIGH,
                         preferred_element_type=_F32) + b_ref[...]
    p = jnp.dot(x_ref[...], wsd_ref[...], precision=_HIGH,
                preferred_element_type=_F32)
    ps_ref[...] = p[:, :_DE]
    pd_ref[...] = p[:, _DE:]


def _proj1(e128, w_bd, b128, x, w_sd):
    return pl.pallas_call(
        _proj1_body,
        grid=(_EB,),
        in_specs=[
            pl.BlockSpec((_EBS, 128), lambda i: (i, 0)),
            pl.BlockSpec((128, 128), lambda i: (0, 0)),
            pl.BlockSpec((1, 128), lambda i: (0, 0)),
            pl.BlockSpec((_NBS, _D), lambda i: (i, 0)),
            pl.BlockSpec((_D, 2 * _DE), lambda i: (0, 0)),
        ],
        out_specs=[
            pl.BlockSpec((_EBS, 128), lambda i: (i, 0)),
            pl.BlockSpec((_NBS, _DE), lambda i: (i, 0)),
            pl.BlockSpec((_NBS, _DE), lambda i: (i, 0)),
        ],
        out_shape=[
            jax.ShapeDtypeStruct((_E8, 128), _F32),
            jax.ShapeDtypeStruct((_N, _DE), _F32),
            jax.ShapeDtypeStruct((_N, _DE), _F32),
        ],
    )(e128, w_bd, b128, x, w_sd)


def _upd2_body(e_ref, wbd_ref, b_ref, x_ref, agg_ref, wnx_ref, wna_ref,
               bn_ref, wsd_ref, t_ref, xo_ref, ps_ref, pd_ref):
    t_ref[...] = jnp.dot(e_ref[...], wbd_ref[...], precision=_HIGH,
                         preferred_element_type=_F32) + b_ref[...]
    agg = agg_ref[0] + agg_ref[1]
    d = (jnp.dot(x_ref[...], wnx_ref[...], precision=_HIGH,
                 preferred_element_type=_F32)
         + jnp.dot(agg, wna_ref[...], precision=_HIGH,
                   preferred_element_type=_F32)
         + bn_ref[...])
    xn = _ALPHA * x_ref[...] + (1.0 - _ALPHA) * jnp.maximum(d, 0.0)
    xo_ref[...] = xn
    p = jnp.dot(xn, wsd_ref[...], precision=_HIGH,
                preferred_element_type=_F32)
    ps_ref[...] = p[:, :_DE]
    pd_ref[...] = p[:, _DE:]


def _upd2(e128, w_bd, b128, x, aggp, wn_x, wn_a, b_n, w_sd):
    return pl.pallas_call(
        _upd2_body,
        grid=(_EB,),
        in_specs=[
            pl.BlockSpec((_EBS, 128), lambda i: (i, 0)),
            pl.BlockSpec((128, 128), lambda i: (0, 0)),
            pl.BlockSpec((1, 128), lambda i: (0, 0)),
            pl.BlockSpec((_NBS, _D), lambda i: (i, 0)),
            pl.BlockSpec((_NC, _NBS, _DE), lambda i: (0, i, 0)),
            pl.BlockSpec((_D, _D), lambda i: (0, 0)),
            pl.BlockSpec((_DE, _D), lambda i: (0, 0)),
            pl.BlockSpec((1, _D), lambda i: (0, 0)),
            pl.BlockSpec((_D, 2 * _DE), lambda i: (0, 0)),
        ],
        out_specs=[
            pl.BlockSpec((_EBS, 128), lambda i: (i, 0)),
            pl.BlockSpec((_NBS, _D), lambda i: (i, 0)),
            pl.BlockSpec((_NBS, _DE), lambda i: (i, 0)),
            pl.BlockSpec((_NBS, _DE), lambda i: (i, 0)),
        ],
        out_shape=[
            jax.ShapeDtypeStruct((_E8, 128), _F32),
            jax.ShapeDtypeStruct((_N, _D), _F32),
            jax.ShapeDtypeStruct((_N, _DE), _F32),
            jax.ShapeDtypeStruct((_N, _DE), _F32),
        ],
    )(e128, w_bd, b128, x, aggp, wn_x, wn_a, b_n.reshape(1, _D), w_sd)


def _nodeupd_body(x_ref, agg_ref, wnx_ref, wna_ref, bn_ref, xo_ref):
    agg = agg_ref[0] + agg_ref[1]
    d = (jnp.dot(x_ref[...], wnx_ref[...], precision=_HIGH,
                 preferred_element_type=_F32)
         + jnp.dot(agg, wna_ref[...], precision=_HIGH,
                   preferred_element_type=_F32)
         + bn_ref[...])
    xo_ref[...] = _ALPHA * x_ref[...] + (1.0 - _ALPHA) * jnp.maximum(d, 0.0)


def _nodeupd(x, aggp, wn_x, wn_a, b_n, bn_blocks=10):
    bn = _N // bn_blocks
    return pl.pallas_call(
        _nodeupd_body,
        grid=(bn_blocks,),
        in_specs=[
            pl.BlockSpec((bn, _D), lambda i: (i, 0)),
            pl.BlockSpec((_NC, bn, _DE), lambda i: (0, i, 0)),
            pl.BlockSpec((_D, _D), lambda i: (0, 0)),
            pl.BlockSpec((_DE, _D), lambda i: (0, 0)),
            pl.BlockSpec((1, _D), lambda i: (0, 0)),
        ],
        out_specs=pl.BlockSpec((bn, _D), lambda i: (i, 0)),
        out_shape=jax.ShapeDtypeStruct((_N, _D), _F32),
    )(x, aggp, wn_x, wn_a, b_n.reshape(1, _D))


_NBUF = 2


def _sc_edge_body(ps_hbm, pd_hbm, t_hbm, src_hbm, dst_hbm,
                  enew_hbm, aggp_hbm,
                  idxr, src_v, dst_v, ps_b, pd_b, t_b, eo_b, zb, agg_sh,
                  sem_ga, sem_gb, sem_wa, sem_wb):
    cid = lax.axis_index("c")
    sid = lax.axis_index("s")
    wid = sid * _NC + cid

    def _zrow(i, _):
        zb[i] = jnp.zeros((16,), _F32)
        return 0
    lax.fori_loop(0, _RPS, _zrow, 0)
    pltpu.sync_copy(zb, agg_sh.at[pl.ds(sid * _RPS, _RPS)])
    plsc.subcore_barrier()

    for b in range(5):
        idxr[pl.ds(16 * b, 16)] = jnp.minimum(
            lax.iota(jnp.int32, 16) * _NW + (512 * b) + wid, _NCHUNK - 1)
    cpi = (pltpu.async_copy(src_hbm.at[idxr], src_v, sem_ga),
           pltpu.async_copy(dst_hbm.at[idxr], dst_v, sem_ga))
    for cp in cpi:
        cp.wait()

    sem_w = (sem_wa, sem_wb)

    def _fire(m, p, sem):
        c = wid + _NW * m
        return (
            pltpu.async_copy(ps_hbm.at[src_v.at[m]], ps_b.at[p], sem),
            pltpu.async_copy(pd_hbm.at[dst_v.at[m]], pd_b.at[p], sem),
            pltpu.async_copy(t_hbm.at[pl.ds(c * _CH8, _CH8)], t_b.at[p], sem),
        )

    def _drain_w(p):
        pltpu.make_async_copy(
            enew_hbm.at[pl.ds(0, _CH)], eo_b.at[p], sem_w[p]).wait()

    def _consume(m, p, cps):
        c = wid + _NW * m
        for cp in cps:
            cp.wait()

        @pl.when(m >= _NBUF)
        def _():
            _drain_w(p)

        def _row(rr, _):
            for a in range(8):
                r = rr * 8 + a
                eo_b[p, r] = jnp.maximum(
                    ps_b[p, r] + pd_b[p, r]
                    + t_b[p, rr, pl.ds(a * _DE, _DE)], 0.0)
            return 0
        lax.fori_loop(0, _CH8, _row, 0)
        pltpu.sync_copy(eo_b.at[p], agg_sh.at[dst_v.at[m]], add=True)
        pltpu.async_copy(eo_b.at[p], enew_hbm.at[pl.ds(c * _CH, _CH)],
                         sem_w[p])

    cps0 = _fire(0, 0, sem_ga)

    def _round(i, _):
        m0 = 2 * i
        cps1 = _fire(m0 + 1, 1, sem_gb)
        _consume(m0, 0, cps0)
        _fire(m0 + 2, 0, sem_ga)
        _consume(m0 + 1, 1, cps1)
        return 0
    lax.fori_loop(0, _MSTEADY // 2, _round, 0)
    for cp in cps0:
        cp.wait()

    @pl.when(wid < _NWEXTRA)
    def _():
        _consume(_MSTEADY, 0, ())
    _drain_w(0)
    _drain_w(1)

    plsc.subcore_barrier()
    pltpu.sync_copy(agg_sh.at[pl.ds(sid * _RPS, _RPS)],
                    aggp_hbm.at[cid, pl.ds(sid * _RPS, _RPS)])


@functools.cache
def _sc_edge():
  return pl.kernel(
    _sc_edge_body,
    out_type=[
        jax.ShapeDtypeStruct((_E, _DE), _F32),
        jax.ShapeDtypeStruct((_NC, _NP, _DE), _F32),
    ],
    mesh=plsc.VectorSubcoreMesh(core_axis_name="c", subcore_axis_name="s",
                                num_cores=_NC, num_subcores=_NS),
    compiler_params=pltpu.CompilerParams(use_tc_tiling_on_sc=False),
    scratch_types=[
        pltpu.VMEM((_MSTEADY + 2,), jnp.int32),
        pltpu.VMEM((_MSTEADY + 2, _CH), jnp.int32),
        pltpu.VMEM((_MSTEADY + 2, _CH), jnp.int32),
        pltpu.VMEM((_NBUF, _CH, _DE), _F32),
        pltpu.VMEM((_NBUF, _CH, _DE), _F32),
        pltpu.VMEM((_NBUF, _CH8, 128), _F32),
        pltpu.VMEM((_NBUF, _CH, _DE), _F32),
        pltpu.VMEM((_RPS, _DE), _F32),
        pltpu.VMEM_SHARED((_NP, _DE), _F32),
        pltpu.SemaphoreType.DMA,
        pltpu.SemaphoreType.DMA,
        pltpu.SemaphoreType.DMA,
        pltpu.SemaphoreType.DMA,
    ],
  )


def kernel(x, edge_index, edge_attr, W_e1, b_e1, W_n1, b_n1,
           W_e2, b_e2, W_n2, b_n2):
    src = edge_index[0].reshape(_NCHUNK, _CH)
    dst = edge_index[1].reshape(_NCHUNK, _CH)

    wsd1 = jnp.concatenate([W_e1[:_D], W_e1[_D:2 * _D]], axis=1)
    we1 = W_e1[2 * _D:]
    wsd2 = jnp.concatenate([W_e2[:_D], W_e2[_D:2 * _D]], axis=1)
    we2 = W_e2[2 * _D:]
    wn1_x, wn1_a = W_n1[:_D], W_n1[_D:]
    wn2_x, wn2_a = W_n2[:_D], W_n2[_D:]

    eye8 = jnp.eye(8, dtype=_F32)
    bd1 = jnp.kron(eye8, we1)
    bd2 = jnp.kron(eye8, we2)
    b1_128 = jnp.tile(b_e1.reshape(1, _DE), (1, 8))
    b2_128 = jnp.tile(b_e2.reshape(1, _DE), (1, 8))

    t1, ps1, pd1 = _proj1(edge_attr.reshape(_E8, 128), bd1, b1_128, x, wsd1)
    e1, aggp1 = _sc_edge()(ps1, pd1, t1, src, dst)

    t2, x1, ps2, pd2 = _upd2(e1.reshape(_E8, 128), bd2, b2_128, x, aggp1,
                             wn1_x, wn1_a, b_n1, wsd2)
    e2, aggp2 = _sc_edge()(ps2, pd2, t2, src, dst)
    x2 = _nodeupd(x1, aggp2, wn2_x, wn2_a, b_n2)

    return (x2, e2, (edge_attr, e1, e2))
